# baseline (speedup 1.0000x reference)
import jax
import jax.numpy as jnp
from jax import lax
from jax.experimental import pallas as pl
from jax.experimental.pallas import tpu as pltpu
from jax.experimental.pallas import tpu_sc as plsc

DIM_ = 100000
ROWS = 256
NT = 782
TK = 17
NCH = NT // TK
PAD = 528


def _sc_body(xpad_hbm, out_hbm, iv_v, sv_v, buf_m, buf_c, sem):
    wid = lax.axis_index("s") * 2 + lax.axis_index("c")
    base_row = wid * 8

    pltpu.sync_copy(xpad_hbm.at[pl.ds(base_row, 16)], iv_v)
    pltpu.sync_copy(xpad_hbm.at[pl.ds(ROWS + base_row, 16)], sv_v)

    lane = lax.broadcasted_iota(jnp.int32, (16,), 0)
    sv = sv_v[...]
    iv = iv_v[...].astype(jnp.int32)
    move_v = (1.0 - jnp.exp(-sv)) * (1.0 / DIM_)
    diag_v = 1.0 - move_v * float(DIM_ - 1)
    ts = [iv[s] // 128 for s in range(8)]

    for s in range(8):
        splat = jnp.full((16,), move_v[s], jnp.float32)

        def fill(k, _):
            for w in range(8):
                buf_m[k, s, pl.ds(w * 16, 16)] = splat
            return 0

        lax.fori_loop(0, TK, fill, 0)

    for sp in range(8):
        splat = jnp.full((16,), move_v[sp], jnp.float32)
        for w in range(8):
            buf_c[0, sp, pl.ds(w * 16, 16)] = splat
    for s in range(8):
        tbase = ts[s] * 128
        for sp in range(8):
            m_sp = jnp.full((16,), move_v[sp], jnp.float32)
            d_sp = jnp.full((16,), diag_v[sp], jnp.float32)
            for w in range(8):
                j = iv[sp] - tbase - w * 16
                buf_c[1 + s, sp, pl.ds(w * 16, 16)] = jnp.where(
                    lane == j, d_sp, m_sp)

    tcs = [t // TK for t in ts]

    def stream_chunk(c, _):
        inside = (tcs[0] == c)
        for s in range(1, 8):
            inside = jnp.logical_or(inside, tcs[s] == c)

        @pl.when(jnp.logical_not(inside))
        def _():
            pltpu.async_copy(buf_m, out_hbm.at[wid, pl.ds(c * TK, TK)], sem)

        @pl.when(inside)
        def _():
            def tile_copy(u, _):
                t = c * TK + u
                sel = jnp.int32(0)
                for s in range(8):
                    sel = jnp.where(t == ts[s], jnp.int32(1 + s), sel)
                pltpu.async_copy(buf_c.at[pl.ds(sel, 1)],
                                 out_hbm.at[wid, pl.ds(t, 1)], sem)
                return 0

            lax.fori_loop(0, TK, tile_copy, 0)

        return 0

    lax.fori_loop(0, NCH, stream_chunk, 0)

    for _ in range(NCH):
        pltpu.make_async_copy(buf_m, out_hbm.at[wid, pl.ds(0, TK)],
                              sem).wait()


def kernel(i, sigma):
    xpad = jnp.zeros((PAD,), jnp.float32)
    xpad = xpad.at[:ROWS].set(i.reshape(ROWS).astype(jnp.float32))
    xpad = xpad.at[ROWS:2 * ROWS].set(sigma.reshape(ROWS))
    mesh = plsc.VectorSubcoreMesh(core_axis_name="c", subcore_axis_name="s")
    run = pl.kernel(
        _sc_body,
        mesh=mesh,
        out_type=jax.ShapeDtypeStruct((32, NT, 8, 128), jnp.float32),
        scratch_types=[
            pltpu.VMEM((16,), jnp.float32),
            pltpu.VMEM((16,), jnp.float32),
            pltpu.VMEM((TK, 8, 128), jnp.float32),
            pltpu.VMEM((9, 8, 128), jnp.float32),
            pltpu.SemaphoreType.DMA,
        ],
    )
    out4 = run(xpad)
    return out4.transpose(0, 2, 1, 3).reshape(32, 8, NT * 128)[:, :, :DIM_]
